# baseline (speedup 1.0000x reference)
import functools

import jax
import jax.numpy as jnp
from jax import lax
from jax.experimental import pallas as pl
from jax.experimental.pallas import tpu as pltpu
from jax.experimental.pallas import tpu_sc as plsc

_L = 16
_K = 128
_G = 10
_WB = 80


def _sc_agg_body(ncores, nsub, N, D, P, E,
                 h_hbm, edges_hbm, out_hbm,
                 src3, dst3, rows0, rows1, ones_v,
                 acc_sh, deg_sh, semg0, semg1, semsc0, semsc1, semi, semd):
    c = lax.axis_index("c")
    s = lax.axis_index("s")
    zero = jnp.zeros((_L,), jnp.float32)
    one = jnp.ones((_L,), jnp.float32)
    rpt = ((N + nsub * _WB - 1) // (nsub * _WB)) * _WB
    nvec = D // _L

    C = E // _K
    NG = C // _G
    grem = NG % nsub
    nrounds = P // ncores

    base = s * rpt
    nrows = jnp.maximum(jnp.minimum(rpt, N - base), 0)
    nwb = (nrows + _WB - 1) // _WB

    wb = rows0.at[pl.ds(0, _WB)]
    degv = ones_v.at[pl.ds(0, _WB)]

    gbufs = (rows0, rows1)
    gsems = (semg0, semg1)
    scsems = (semsc0, semsc1)

    for r in range(nrounds):
        m = c * nrounds + r

        def fill_z(i, _):
            for j in range(nvec):
                rows0[i, pl.ds(j * _L, _L)] = zero
            ones_v[i, :] = zero
            return 0
        lax.fori_loop(0, _WB, fill_z, 0)

        def zbody(j, _):
            r0 = base + j * _WB
            pltpu.sync_copy(wb, acc_sh.at[pl.ds(r0, _WB)])
            pltpu.sync_copy(degv, deg_sh.at[pl.ds(r0, _WB)])
            return 0
        lax.fori_loop(0, nwb, zbody, 0)

        def fill_ones(i, _):
            ones_v[i, :] = one
            return 0
        lax.fori_loop(0, _K, fill_ones, 0)
        plsc.subcore_barrier()

        ngrp = (NG // nsub) + jnp.where(s < grem, 1, 0)

        pltpu.async_copy(edges_hbm.at[2 * m, pl.ds(s * _G, _G)],
                         src3.at[0], semi)
        pltpu.async_copy(edges_hbm.at[2 * m + 1, pl.ds(s * _G, _G)],
                         dst3.at[0], semi)

        def grp(i, _):
            ip = lax.rem(i, 2)
            pltpu.make_async_copy(edges_hbm.at[2 * m, pl.ds(0, _G)],
                                  src3.at[0], semi).wait()
            pltpu.make_async_copy(edges_hbm.at[2 * m, pl.ds(0, _G)],
                                  dst3.at[0], semi).wait()
            gin = jnp.minimum(s + (i + 1) * nsub, NG - 1)
            pltpu.async_copy(edges_hbm.at[2 * m, pl.ds(gin * _G, _G)],
                             src3.at[1 - ip], semi)
            pltpu.async_copy(edges_hbm.at[2 * m + 1, pl.ds(gin * _G, _G)],
                             dst3.at[1 - ip], semi)
            sc = [None] * _G
            dg = [None] * _G
            pending = pltpu.async_copy(h_hbm.at[src3.at[ip, 0]], rows0,
                                       semg0)
            for j in range(_G):
                if j + 1 < _G:
                    if j >= 1:
                        sc[j - 1].wait()
                    nxt = pltpu.async_copy(h_hbm.at[src3.at[ip, j + 1]],
                                           gbufs[(j + 1) % 2],
                                           gsems[(j + 1) % 2])
                pending.wait()
                sc[j] = pltpu.async_copy(gbufs[j % 2],
                                         acc_sh.at[dst3.at[ip, j]],
                                         scsems[j % 2], add=True)
                dg[j] = pltpu.async_copy(ones_v, deg_sh.at[dst3.at[ip, j]],
                                         semd, add=True)
                if j + 1 < _G:
                    pending = nxt
            sc[_G - 2].wait()
            sc[_G - 1].wait()
            for j in range(_G):
                dg[j].wait()
            return 0
        lax.fori_loop(0, ngrp, grp, 0)
        pltpu.make_async_copy(edges_hbm.at[2 * m, pl.ds(0, _G)],
                              src3.at[0], semi).wait()
        pltpu.make_async_copy(edges_hbm.at[2 * m, pl.ds(0, _G)],
                              dst3.at[0], semi).wait()
        plsc.subcore_barrier()

        def wbody(j, _):
            r0 = base + j * _WB
            pltpu.sync_copy(acc_sh.at[pl.ds(r0, _WB)], wb)
            pltpu.sync_copy(deg_sh.at[pl.ds(r0, _WB)], degv)

            def rbody(i, _):
                rcp = 1.0 / (degv[i, :] + 1e-8)
                for jj in range(nvec):
                    rows0[i, pl.ds(jj * _L, _L)] = (
                        rows0[i, pl.ds(jj * _L, _L)] * rcp)
                return 0
            lax.fori_loop(0, _WB, rbody, 0)
            pltpu.sync_copy(wb, out_hbm.at[m, pl.ds(r0, _WB)])
            return 0
        lax.fori_loop(0, nwb, wbody, 0)
        plsc.subcore_barrier()


def _sc_aggregate(h, edges2, P):
    N, D = h.shape
    E = edges2.shape[1] * edges2.shape[2]
    info = plsc.get_sparse_core_info()
    nc, ns = info.num_cores, info.num_subcores
    body = functools.partial(_sc_agg_body, nc, ns, N, D, P, E)
    mesh = plsc.VectorSubcoreMesh(core_axis_name="c", subcore_axis_name="s")
    f = pl.kernel(
        body,
        out_type=jax.ShapeDtypeStruct((P, N, D), jnp.float32),
        mesh=mesh,
        scratch_types=[
            pltpu.VMEM((2, _G, _K), jnp.int32),
            pltpu.VMEM((2, _G, _K), jnp.int32),
            pltpu.VMEM((_K, D), jnp.float32),
            pltpu.VMEM((_K, D), jnp.float32),
            pltpu.VMEM((_K, _L), jnp.float32),
            pltpu.VMEM_SHARED((N, D), jnp.float32),
            pltpu.VMEM_SHARED((N, _L), jnp.float32),
            pltpu.SemaphoreType.DMA,
            pltpu.SemaphoreType.DMA,
            pltpu.SemaphoreType.DMA,
            pltpu.SemaphoreType.DMA,
            pltpu.SemaphoreType.DMA,
            pltpu.SemaphoreType.DMA,
        ],
        compiler_params=pltpu.CompilerParams(use_tc_tiling_on_sc=False),
    )
    return f(h, edges2)


def _t1_body(macc_ref, w_ref, a_ref, fcw_ref, fcb_ref, e_ref, s_ref):
    m = pl.program_id(0)
    n = pl.program_id(1)
    x = macc_ref[0]
    w = w_ref[0]
    y = lax.dot_general(x, w, (((1,), (1,)), ((), ())),
                        preferred_element_type=jnp.float32)
    a = a_ref[m]
    e = jnp.where(y >= 0.0, y, a * y)
    e_ref[0] = e
    t = jnp.tanh(lax.dot_general(e, fcw_ref[...], (((1,), (1,)), ((), ())),
                                 preferred_element_type=jnp.float32)
                 + fcb_ref[...])

    @pl.when(n == 0)
    def _():
        s_ref[...] = jnp.zeros_like(s_ref)
    s_ref[0, 0] += jnp.sum(t, axis=0)


def _t2_body(n_total, s_ref, att_ref, e_ref, z_ref):
    sp = s_ref[...][:, 0, :] * (1.0 / n_total)
    logits = jnp.sum(sp * att_ref[...], axis=1)
    mx = jnp.max(logits)
    ew = jnp.exp(logits - mx)
    beta = ew / jnp.sum(ew)
    z_ref[...] = jnp.sum(beta[:, None, None] * e_ref[...], axis=0)


def kernel(h, edge_indices, W_agg, prelu_a, fc_W, fc_b, att):
    N, D = h.shape
    P = edge_indices.shape[0]
    E = edge_indices.shape[2]
    edges2 = edge_indices.reshape(2 * P, E // _K, _K)

    macc = _sc_aggregate(h, edges2, P)

    nb = 10
    bn = N // nb
    e, S = pl.pallas_call(
        _t1_body,
        grid=(P, nb),
        in_specs=[
            pl.BlockSpec((1, bn, D), lambda m, n: (m, n, 0)),
            pl.BlockSpec((1, D, D), lambda m, n: (m, 0, 0)),
            pl.BlockSpec(memory_space=pltpu.SMEM),
            pl.BlockSpec((D, D), lambda m, n: (0, 0)),
            pl.BlockSpec((1, D), lambda m, n: (0, 0)),
        ],
        out_specs=[
            pl.BlockSpec((1, bn, D), lambda m, n: (m, n, 0)),
            pl.BlockSpec((1, 1, D), lambda m, n: (m, 0, 0)),
        ],
        out_shape=[
            jax.ShapeDtypeStruct((P, N, D), jnp.float32),
            jax.ShapeDtypeStruct((P, 1, D), jnp.float32),
        ],
    )(macc, W_agg, prelu_a, fc_W, fc_b.reshape(1, D))

    z = pl.pallas_call(
        functools.partial(_t2_body, N),
        grid=(nb,),
        in_specs=[
            pl.BlockSpec((P, 1, D), lambda n: (0, 0, 0)),
            pl.BlockSpec((1, D), lambda n: (0, 0)),
            pl.BlockSpec((P, bn, D), lambda n: (0, n, 0)),
        ],
        out_specs=pl.BlockSpec((bn, D), lambda n: (n, 0)),
        out_shape=jax.ShapeDtypeStruct((N, D), jnp.float32),
    )(S, att, e)
    return z


# speedup vs baseline: 1.0293x; 1.0293x over previous
import functools

import jax
import jax.numpy as jnp
from jax import lax
from jax.experimental import pallas as pl
from jax.experimental.pallas import tpu as pltpu
from jax.experimental.pallas import tpu_sc as plsc

_L = 16
_K = 128
_G = 10
_WB = 80


def _sc_agg_body(ncores, nsub, N, D, E, mbase,
                 h_hbm, edges_hbm, out_hbm,
                 src3, dst3, rows0, rows1, ones_v,
                 acc_sh, deg_sh, semg0, semg1, semsc0, semsc1, semi, semd):
    c = lax.axis_index("c")
    s = lax.axis_index("s")
    zero = jnp.zeros((_L,), jnp.float32)
    one = jnp.ones((_L,), jnp.float32)
    rpt = ((N + nsub * _WB - 1) // (nsub * _WB)) * _WB
    nvec = D // _L

    C = E // _K
    NG = C // _G
    grem = NG % nsub

    base = s * rpt
    nrows = jnp.maximum(jnp.minimum(rpt, N - base), 0)
    nwb = (nrows + _WB - 1) // _WB

    wb = rows0.at[pl.ds(0, _WB)]
    degv = ones_v.at[pl.ds(0, _WB)]

    gbufs = (rows0, rows1)
    gsems = (semg0, semg1)
    scsems = (semsc0, semsc1)

    for r in range(1):
        m = mbase + c

        def fill_z(i, _):
            for j in range(nvec):
                rows0[i, pl.ds(j * _L, _L)] = zero
            ones_v[i, :] = zero
            return 0
        lax.fori_loop(0, _WB, fill_z, 0)

        def zbody(j, _):
            r0 = base + j * _WB
            pltpu.sync_copy(wb, acc_sh.at[pl.ds(r0, _WB)])
            pltpu.sync_copy(degv, deg_sh.at[pl.ds(r0, _WB)])
            return 0
        lax.fori_loop(0, nwb, zbody, 0)

        def fill_ones(i, _):
            ones_v[i, :] = one
            return 0
        lax.fori_loop(0, _K, fill_ones, 0)
        plsc.subcore_barrier()

        ngrp = (NG // nsub) + jnp.where(s < grem, 1, 0)

        pltpu.async_copy(edges_hbm.at[2 * m, pl.ds(s * _G, _G)],
                         src3.at[0], semi)
        pltpu.async_copy(edges_hbm.at[2 * m + 1, pl.ds(s * _G, _G)],
                         dst3.at[0], semi)

        def grp(i, _):
            ip = lax.rem(i, 2)
            pltpu.make_async_copy(edges_hbm.at[2 * m, pl.ds(0, _G)],
                                  src3.at[0], semi).wait()
            pltpu.make_async_copy(edges_hbm.at[2 * m, pl.ds(0, _G)],
                                  dst3.at[0], semi).wait()
            gin = jnp.minimum(s + (i + 1) * nsub, NG - 1)
            pltpu.async_copy(edges_hbm.at[2 * m, pl.ds(gin * _G, _G)],
                             src3.at[1 - ip], semi)
            pltpu.async_copy(edges_hbm.at[2 * m + 1, pl.ds(gin * _G, _G)],
                             dst3.at[1 - ip], semi)
            sc = [None] * _G
            dg = [None] * _G
            pending = pltpu.async_copy(h_hbm.at[src3.at[ip, 0]], rows0,
                                       semg0)
            for j in range(_G):
                if j + 1 < _G:
                    if j >= 1:
                        sc[j - 1].wait()
                    nxt = pltpu.async_copy(h_hbm.at[src3.at[ip, j + 1]],
                                           gbufs[(j + 1) % 2],
                                           gsems[(j + 1) % 2])
                pending.wait()
                sc[j] = pltpu.async_copy(gbufs[j % 2],
                                         acc_sh.at[dst3.at[ip, j]],
                                         scsems[j % 2], add=True)
                dg[j] = pltpu.async_copy(ones_v, deg_sh.at[dst3.at[ip, j]],
                                         semd, add=True)
                if j + 1 < _G:
                    pending = nxt
            sc[_G - 2].wait()
            sc[_G - 1].wait()
            for j in range(_G):
                dg[j].wait()
            return 0
        lax.fori_loop(0, ngrp, grp, 0)
        pltpu.make_async_copy(edges_hbm.at[2 * m, pl.ds(0, _G)],
                              src3.at[0], semi).wait()
        pltpu.make_async_copy(edges_hbm.at[2 * m, pl.ds(0, _G)],
                              dst3.at[0], semi).wait()
        plsc.subcore_barrier()

        def wbody(j, _):
            r0 = base + j * _WB
            pltpu.sync_copy(acc_sh.at[pl.ds(r0, _WB)], wb)
            pltpu.sync_copy(deg_sh.at[pl.ds(r0, _WB)], degv)

            def rbody(i, _):
                rcp = 1.0 / (degv[i, :] + 1e-8)
                for jj in range(nvec):
                    rows0[i, pl.ds(jj * _L, _L)] = (
                        rows0[i, pl.ds(jj * _L, _L)] * rcp)
                return 0
            lax.fori_loop(0, _WB, rbody, 0)
            pltpu.sync_copy(wb, out_hbm.at[c, pl.ds(r0, _WB)])
            return 0
        lax.fori_loop(0, nwb, wbody, 0)
        plsc.subcore_barrier()


def _sc_aggregate(h, edges2, mbase):
    N, D = h.shape
    E = edges2.shape[1] * edges2.shape[2]
    info = plsc.get_sparse_core_info()
    nc, ns = info.num_cores, info.num_subcores
    body = functools.partial(_sc_agg_body, nc, ns, N, D, E, mbase)
    mesh = plsc.VectorSubcoreMesh(core_axis_name="c", subcore_axis_name="s")
    f = pl.kernel(
        body,
        out_type=jax.ShapeDtypeStruct((nc, N, D), jnp.float32),
        mesh=mesh,
        scratch_types=[
            pltpu.VMEM((2, _G, _K), jnp.int32),
            pltpu.VMEM((2, _G, _K), jnp.int32),
            pltpu.VMEM((_K, D), jnp.float32),
            pltpu.VMEM((_K, D), jnp.float32),
            pltpu.VMEM((_K, _L), jnp.float32),
            pltpu.VMEM_SHARED((N, D), jnp.float32),
            pltpu.VMEM_SHARED((N, _L), jnp.float32),
            pltpu.SemaphoreType.DMA,
            pltpu.SemaphoreType.DMA,
            pltpu.SemaphoreType.DMA,
            pltpu.SemaphoreType.DMA,
            pltpu.SemaphoreType.DMA,
            pltpu.SemaphoreType.DMA,
        ],
        compiler_params=pltpu.CompilerParams(use_tc_tiling_on_sc=False),
    )
    return f(h, edges2)


def _t1_body(macc_ref, w_ref, a_ref, fcw_ref, fcb_ref, s_ref, *, mb):
    m = pl.program_id(0)
    n = pl.program_id(1)
    x = macc_ref[0]
    w = w_ref[0]
    y = lax.dot_general(x, w, (((1,), (1,)), ((), ())),
                        preferred_element_type=jnp.float32)
    a = a_ref[mb + m]
    e = jnp.where(y >= 0.0, y, a * y)
    t = jnp.tanh(lax.dot_general(e, fcw_ref[...], (((1,), (1,)), ((), ())),
                                 preferred_element_type=jnp.float32)
                 + fcb_ref[...])

    @pl.when(n == 0)
    def _():
        s_ref[...] = jnp.zeros_like(s_ref)
    s_ref[0, 0] += jnp.sum(t, axis=0)


def _t2_body(n_total, sa_ref, sb_ref, att_ref, ma_ref, mb_ref, w_ref, a_ref,
             z_ref):
    sp = jnp.concatenate([sa_ref[...], sb_ref[...]], axis=0)[:, 0, :]
    sp = sp * (1.0 / n_total)
    logits = jnp.sum(sp * att_ref[...], axis=1)
    mx = jnp.max(logits)
    ew = jnp.exp(logits - mx)
    beta = ew / jnp.sum(ew)
    acc = jnp.zeros_like(z_ref)
    for m in range(4):
        src = ma_ref if m < 2 else mb_ref
        y = lax.dot_general(src[m % 2], w_ref[m], (((1,), (1,)), ((), ())),
                            preferred_element_type=jnp.float32)
        e = jnp.where(y >= 0.0, y, a_ref[m] * y)
        acc = acc + beta[m] * e
    z_ref[...] = acc


def _t1(macc, W_agg, prelu_a, fc_W, fc_b2, mb, nb, bn, N, D):
    return pl.pallas_call(
        functools.partial(_t1_body, mb=mb),
        grid=(2, nb),
        in_specs=[
            pl.BlockSpec((1, bn, D), lambda m, n: (m, n, 0)),
            pl.BlockSpec((1, D, D), lambda m, n, _mb=mb: (m + _mb, 0, 0)),
            pl.BlockSpec(memory_space=pltpu.SMEM),
            pl.BlockSpec((D, D), lambda m, n: (0, 0)),
            pl.BlockSpec((1, D), lambda m, n: (0, 0)),
        ],
        out_specs=pl.BlockSpec((1, 1, D), lambda m, n: (m, 0, 0)),
        out_shape=jax.ShapeDtypeStruct((2, 1, D), jnp.float32),
    )(macc, W_agg, prelu_a, fc_W, fc_b2)


def kernel(h, edge_indices, W_agg, prelu_a, fc_W, fc_b, att):
    N, D = h.shape
    P = edge_indices.shape[0]
    E = edge_indices.shape[2]
    edges2 = edge_indices.reshape(2 * P, E // _K, _K)

    macc_a = _sc_aggregate(h, edges2, 0)
    macc_b = _sc_aggregate(h, edges2, 2)

    nb = 10
    bn = N // nb
    fc_b2 = fc_b.reshape(1, D)
    s_a = _t1(macc_a, W_agg, prelu_a, fc_W, fc_b2, 0, nb, bn, N, D)
    s_b = _t1(macc_b, W_agg, prelu_a, fc_W, fc_b2, 2, nb, bn, N, D)

    z = pl.pallas_call(
        functools.partial(_t2_body, N),
        grid=(nb,),
        in_specs=[
            pl.BlockSpec((2, 1, D), lambda n: (0, 0, 0)),
            pl.BlockSpec((2, 1, D), lambda n: (0, 0, 0)),
            pl.BlockSpec((1, D), lambda n: (0, 0)),
            pl.BlockSpec((2, bn, D), lambda n: (0, n, 0)),
            pl.BlockSpec((2, bn, D), lambda n: (0, n, 0)),
            pl.BlockSpec((P, D, D), lambda n: (0, 0, 0)),
            pl.BlockSpec(memory_space=pltpu.SMEM),
        ],
        out_specs=pl.BlockSpec((bn, D), lambda n: (n, 0)),
        out_shape=jax.ShapeDtypeStruct((N, D), jnp.float32),
    )(s_a, s_b, att, macc_a, macc_b, W_agg, prelu_a)
    return z


# speedup vs baseline: 1.0736x; 1.0431x over previous
import functools

import jax
import jax.numpy as jnp
from jax import lax
from jax.experimental import pallas as pl
from jax.experimental.pallas import tpu as pltpu
from jax.experimental.pallas import tpu_sc as plsc

_L = 16
_K = 128
_G = 10
_WB = 80


def _sc_agg_body(ncores, nsub, N, D, E, mbase,
                 h_hbm, edges_hbm, out_hbm,
                 src3, dst3, rows0, rows1, hist, iota_idx, degz,
                 acc_sh, deg_sh, semg0, semg1, semsc0, semsc1, semi, semd):
    c = lax.axis_index("c")
    s = lax.axis_index("s")
    zero = jnp.zeros((_L,), jnp.float32)
    one = jnp.ones((_L,), jnp.float32)
    rpt = ((N + nsub * _WB - 1) // (nsub * _WB)) * _WB
    nvec = D // _L

    C = E // _K
    NG = C // _G
    grem = NG % nsub

    base = s * rpt
    nrows = jnp.maximum(jnp.minimum(rpt, N - base), 0)
    nwb = (nrows + _WB - 1) // _WB

    wb = rows0.at[pl.ds(0, _WB)]
    nhr = (rpt * nsub) // _L
    hpt = nhr // nsub
    iota16 = lax.iota(jnp.int32, _L)

    for hr in range(nhr // _K):
        for k in range(_K // _L):
            iota_idx[hr, pl.ds(k * _L, _L)] = iota16 + (hr * _K + k * _L)

    gbufs = (rows0, rows1)
    gsems = (semg0, semg1)
    scsems = (semsc0, semsc1)

    for r in range(1):
        m = mbase + c

        def fill_z(i, _):
            for j in range(nvec):
                rows0[i, pl.ds(j * _L, _L)] = zero
            return 0
        lax.fori_loop(0, _WB, fill_z, 0)

        def fill_dz(i, _):
            degz[i, :] = zero
            return 0
        lax.fori_loop(0, hpt, fill_dz, 0)

        def fill_h(i, _):
            hist[i, :] = zero
            return 0
        lax.fori_loop(0, nhr, fill_h, 0)

        def zbody(j, _):
            r0 = base + j * _WB
            pltpu.sync_copy(wb, acc_sh.at[pl.ds(r0, _WB)])
            return 0
        lax.fori_loop(0, nwb, zbody, 0)
        pltpu.sync_copy(degz, deg_sh.at[pl.ds(s * hpt, hpt)])
        plsc.subcore_barrier()

        ngrp = (NG // nsub) + jnp.where(s < grem, 1, 0)

        pltpu.async_copy(edges_hbm.at[2 * m, pl.ds(s * _G, _G)],
                         src3.at[0], semi)
        pltpu.async_copy(edges_hbm.at[2 * m + 1, pl.ds(s * _G, _G)],
                         dst3.at[0], semi)

        def grp(i, _):
            ip = lax.rem(i, 2)
            pltpu.make_async_copy(edges_hbm.at[2 * m, pl.ds(0, _G)],
                                  src3.at[0], semi).wait()
            pltpu.make_async_copy(edges_hbm.at[2 * m, pl.ds(0, _G)],
                                  dst3.at[0], semi).wait()
            gin = jnp.minimum(s + (i + 1) * nsub, NG - 1)
            pltpu.async_copy(edges_hbm.at[2 * m, pl.ds(gin * _G, _G)],
                             src3.at[1 - ip], semi)
            pltpu.async_copy(edges_hbm.at[2 * m + 1, pl.ds(gin * _G, _G)],
                             dst3.at[1 - ip], semi)
            sc = [None] * _G
            pending = pltpu.async_copy(h_hbm.at[src3.at[ip, 0]], rows0,
                                       semg0)
            for j in range(_G):
                if j + 1 < _G:
                    if j >= 1:
                        sc[j - 1].wait()
                    nxt = pltpu.async_copy(h_hbm.at[src3.at[ip, j + 1]],
                                           gbufs[(j + 1) % 2],
                                           gsems[(j + 1) % 2])
                pending.wait()
                sc[j] = pltpu.async_copy(gbufs[j % 2],
                                         acc_sh.at[dst3.at[ip, j]],
                                         scsems[j % 2], add=True)
                for k in range(_K // _L):
                    dv = dst3[ip, j, pl.ds(k * _L, _L)]
                    ri = lax.shift_right_logical(dv, 4)
                    ci = lax.bitwise_and(dv, 15)
                    plsc.addupdate_scatter(hist, [ri, ci], one)
                if j + 1 < _G:
                    pending = nxt
            sc[_G - 2].wait()
            sc[_G - 1].wait()
            return 0
        lax.fori_loop(0, ngrp, grp, 0)
        pltpu.make_async_copy(edges_hbm.at[2 * m, pl.ds(0, _G)],
                              src3.at[0], semi).wait()
        pltpu.make_async_copy(edges_hbm.at[2 * m, pl.ds(0, _G)],
                              dst3.at[0], semi).wait()
        dgc = []
        for hr in range(nhr // _K):
            dgc.append(pltpu.async_copy(hist.at[pl.ds(hr * _K, _K)],
                                        deg_sh.at[iota_idx.at[hr]],
                                        semd, add=True))
        for d in dgc:
            d.wait()
        plsc.subcore_barrier()

        def wbody(j, _):
            r0 = base + j * _WB
            pltpu.sync_copy(acc_sh.at[pl.ds(r0, _WB)], wb)
            pltpu.sync_copy(deg_sh.at[pl.ds(r0 // _L, _WB // _L)],
                            degz.at[pl.ds(0, _WB // _L)])

            def rbody(g, _):
                rcpv = 1.0 / (degz[g, :] + 1e-8)
                for k in range(_L):
                    rcp = jnp.broadcast_to(rcpv[k], (_L,))
                    i = g * _L + k
                    for jj in range(nvec):
                        rows0[i, pl.ds(jj * _L, _L)] = (
                            rows0[i, pl.ds(jj * _L, _L)] * rcp)
                return 0
            lax.fori_loop(0, _WB // _L, rbody, 0)
            pltpu.sync_copy(wb, out_hbm.at[c, pl.ds(r0, _WB)])
            return 0
        lax.fori_loop(0, nwb, wbody, 0)
        plsc.subcore_barrier()


def _sc_aggregate(h, edges2, mbase):
    N, D = h.shape
    E = edges2.shape[1] * edges2.shape[2]
    info = plsc.get_sparse_core_info()
    nc, ns = info.num_cores, info.num_subcores
    body = functools.partial(_sc_agg_body, nc, ns, N, D, E, mbase)
    mesh = plsc.VectorSubcoreMesh(core_axis_name="c", subcore_axis_name="s")
    f = pl.kernel(
        body,
        out_type=jax.ShapeDtypeStruct((nc, N, D), jnp.float32),
        mesh=mesh,
        scratch_types=[
            pltpu.VMEM((2, _G, _K), jnp.int32),
            pltpu.VMEM((2, _G, _K), jnp.int32),
            pltpu.VMEM((_K, D), jnp.float32),
            pltpu.VMEM((_K, D), jnp.float32),
            pltpu.VMEM((640, _L), jnp.float32),
            pltpu.VMEM((5, _K), jnp.int32),
            pltpu.VMEM((40, _L), jnp.float32),
            pltpu.VMEM_SHARED((N, D), jnp.float32),
            pltpu.VMEM_SHARED((640, _L), jnp.float32),
            pltpu.SemaphoreType.DMA,
            pltpu.SemaphoreType.DMA,
            pltpu.SemaphoreType.DMA,
            pltpu.SemaphoreType.DMA,
            pltpu.SemaphoreType.DMA,
            pltpu.SemaphoreType.DMA,
        ],
        compiler_params=pltpu.CompilerParams(use_tc_tiling_on_sc=False,
                                             needs_layout_passes=False),
    )
    return f(h, edges2)


def _t1_body(macc_ref, w_ref, a_ref, fcw_ref, fcb_ref, s_ref, *, mb):
    m = pl.program_id(0)
    n = pl.program_id(1)
    x = macc_ref[0]
    w = w_ref[0]
    y = lax.dot_general(x, w, (((1,), (1,)), ((), ())),
                        preferred_element_type=jnp.float32)
    a = a_ref[mb + m]
    e = jnp.where(y >= 0.0, y, a * y)
    t = jnp.tanh(lax.dot_general(e, fcw_ref[...], (((1,), (1,)), ((), ())),
                                 preferred_element_type=jnp.float32)
                 + fcb_ref[...])

    @pl.when(n == 0)
    def _():
        s_ref[...] = jnp.zeros_like(s_ref)
    s_ref[0, 0] += jnp.sum(t, axis=0)


def _t2_body(n_total, sa_ref, sb_ref, att_ref, ma_ref, mb_ref, w_ref, a_ref,
             z_ref):
    sp = jnp.concatenate([sa_ref[...], sb_ref[...]], axis=0)[:, 0, :]
    sp = sp * (1.0 / n_total)
    logits = jnp.sum(sp * att_ref[...], axis=1)
    mx = jnp.max(logits)
    ew = jnp.exp(logits - mx)
    beta = ew / jnp.sum(ew)
    acc = jnp.zeros_like(z_ref)
    for m in range(4):
        src = ma_ref if m < 2 else mb_ref
        y = lax.dot_general(src[m % 2], w_ref[m], (((1,), (1,)), ((), ())),
                            preferred_element_type=jnp.float32)
        e = jnp.where(y >= 0.0, y, a_ref[m] * y)
        acc = acc + beta[m] * e
    z_ref[...] = acc


def _t1(macc, W_agg, prelu_a, fc_W, fc_b2, mb, nb, bn, N, D):
    return pl.pallas_call(
        functools.partial(_t1_body, mb=mb),
        grid=(2, nb),
        in_specs=[
            pl.BlockSpec((1, bn, D), lambda m, n: (m, n, 0)),
            pl.BlockSpec((1, D, D), lambda m, n, _mb=mb: (m + _mb, 0, 0)),
            pl.BlockSpec(memory_space=pltpu.SMEM),
            pl.BlockSpec((D, D), lambda m, n: (0, 0)),
            pl.BlockSpec((1, D), lambda m, n: (0, 0)),
        ],
        out_specs=pl.BlockSpec((1, 1, D), lambda m, n: (m, 0, 0)),
        out_shape=jax.ShapeDtypeStruct((2, 1, D), jnp.float32),
    )(macc, W_agg, prelu_a, fc_W, fc_b2)


def kernel(h, edge_indices, W_agg, prelu_a, fc_W, fc_b, att):
    N, D = h.shape
    P = edge_indices.shape[0]
    E = edge_indices.shape[2]
    edges2 = edge_indices.reshape(2 * P, E // _K, _K)

    macc_a = _sc_aggregate(h, edges2, 0)
    macc_b = _sc_aggregate(h, edges2, 2)

    nb = 10
    bn = N // nb
    fc_b2 = fc_b.reshape(1, D)
    s_a = _t1(macc_a, W_agg, prelu_a, fc_W, fc_b2, 0, nb, bn, N, D)
    s_b = _t1(macc_b, W_agg, prelu_a, fc_W, fc_b2, 2, nb, bn, N, D)

    z = pl.pallas_call(
        functools.partial(_t2_body, N),
        grid=(nb,),
        in_specs=[
            pl.BlockSpec((2, 1, D), lambda n: (0, 0, 0)),
            pl.BlockSpec((2, 1, D), lambda n: (0, 0, 0)),
            pl.BlockSpec((1, D), lambda n: (0, 0)),
            pl.BlockSpec((2, bn, D), lambda n: (0, n, 0)),
            pl.BlockSpec((2, bn, D), lambda n: (0, n, 0)),
            pl.BlockSpec((P, D, D), lambda n: (0, 0, 0)),
            pl.BlockSpec(memory_space=pltpu.SMEM),
        ],
        out_specs=pl.BlockSpec((bn, D), lambda n: (n, 0)),
        out_shape=jax.ShapeDtypeStruct((N, D), jnp.float32),
    )(s_a, s_b, att, macc_a, macc_b, W_agg, prelu_a)
    return z


# speedup vs baseline: 1.0852x; 1.0108x over previous
import functools

import jax
import jax.numpy as jnp
from jax import lax
from jax.experimental import pallas as pl
from jax.experimental.pallas import tpu as pltpu
from jax.experimental.pallas import tpu_sc as plsc

_L = 16
_K = 128
_G = 10
_WB = 80


def _sc_agg_body(ncores, nsub, N, D, E, mbase,
                 h_hbm, edges_hbm, out_hbm,
                 src3, dst3, rows0, rows1, hist, iota_idx, degz,
                 acc_sh, deg_sh, semg0, semg1, semsc0, semsc1, semi, semd):
    c = lax.axis_index("c")
    s = lax.axis_index("s")
    zero = jnp.zeros((_L,), jnp.float32)
    one = jnp.ones((_L,), jnp.float32)
    rpt = ((N + nsub * _WB - 1) // (nsub * _WB)) * _WB
    nvec = D // _L

    C = E // _K
    NG = C // _G
    grem = NG % nsub

    base = s * rpt
    nrows = jnp.maximum(jnp.minimum(rpt, N - base), 0)
    nwb = (nrows + _WB - 1) // _WB

    wb = rows0.at[pl.ds(0, _WB)]
    nhr = (rpt * nsub) // _L
    hpt = nhr // nsub
    iota16 = lax.iota(jnp.int32, _L)

    for hr in range(nhr // _K):
        for k in range(_K // _L):
            iota_idx[hr, pl.ds(k * _L, _L)] = iota16 + (hr * _K + k * _L)

    gbufs = (rows0, rows1)
    gsems = (semg0, semg1)
    scsems = (semsc0, semsc1)

    for r in range(1):
        m = mbase + c

        def fill_z(i, _):
            for j in range(nvec):
                rows0[i, pl.ds(j * _L, _L)] = zero
            return 0
        lax.fori_loop(0, _WB, fill_z, 0)

        def fill_dz(i, _):
            degz[i, :] = zero
            return 0
        lax.fori_loop(0, hpt, fill_dz, 0)

        def fill_h(i, _):
            for kk in range(8):
                hist[i * 8 + kk, :] = zero
            return 0
        lax.fori_loop(0, nhr // 8, fill_h, 0)

        def zbody(j, _):
            r0 = base + j * _WB
            pltpu.sync_copy(wb, acc_sh.at[pl.ds(r0, _WB)])
            return 0
        lax.fori_loop(0, nwb, zbody, 0)
        pltpu.sync_copy(degz, deg_sh.at[pl.ds(s * hpt, hpt)])
        plsc.subcore_barrier()

        ngrp = (NG // nsub) + jnp.where(s < grem, 1, 0)

        pltpu.async_copy(edges_hbm.at[2 * m, pl.ds(s * _G, _G)],
                         src3.at[0], semi)
        pltpu.async_copy(edges_hbm.at[2 * m + 1, pl.ds(s * _G, _G)],
                         dst3.at[0], semi)

        def grp(i, _):
            ip = lax.rem(i, 2)
            pltpu.make_async_copy(edges_hbm.at[2 * m, pl.ds(0, _G)],
                                  src3.at[0], semi).wait()
            pltpu.make_async_copy(edges_hbm.at[2 * m, pl.ds(0, _G)],
                                  dst3.at[0], semi).wait()
            gin = jnp.minimum(s + (i + 1) * nsub, NG - 1)
            pltpu.async_copy(edges_hbm.at[2 * m, pl.ds(gin * _G, _G)],
                             src3.at[1 - ip], semi)
            pltpu.async_copy(edges_hbm.at[2 * m + 1, pl.ds(gin * _G, _G)],
                             dst3.at[1 - ip], semi)
            sc = [None] * _G
            pending = pltpu.async_copy(h_hbm.at[src3.at[ip, 0]], rows0,
                                       semg0)
            for j in range(_G):
                if j + 1 < _G:
                    if j >= 1:
                        sc[j - 1].wait()
                    nxt = pltpu.async_copy(h_hbm.at[src3.at[ip, j + 1]],
                                           gbufs[(j + 1) % 2],
                                           gsems[(j + 1) % 2])
                pending.wait()
                sc[j] = pltpu.async_copy(gbufs[j % 2],
                                         acc_sh.at[dst3.at[ip, j]],
                                         scsems[j % 2], add=True)
                for k in range(_K // _L):
                    dv = dst3[ip, j, pl.ds(k * _L, _L)]
                    ri = lax.shift_right_logical(dv, 4)
                    ci = lax.bitwise_and(dv, 15)
                    plsc.addupdate_scatter(hist, [ri, ci], one)
                if j + 1 < _G:
                    pending = nxt
            sc[_G - 2].wait()
            sc[_G - 1].wait()
            return 0
        lax.fori_loop(0, ngrp, grp, 0)
        pltpu.make_async_copy(edges_hbm.at[2 * m, pl.ds(0, _G)],
                              src3.at[0], semi).wait()
        pltpu.make_async_copy(edges_hbm.at[2 * m, pl.ds(0, _G)],
                              dst3.at[0], semi).wait()
        dgc = []
        for hr in range(nhr // _K):
            dgc.append(pltpu.async_copy(hist.at[pl.ds(hr * _K, _K)],
                                        deg_sh.at[iota_idx.at[hr]],
                                        semd, add=True))
        for d in dgc:
            d.wait()
        plsc.subcore_barrier()

        def wbody(j, _):
            r0 = base + j * _WB
            pltpu.sync_copy(acc_sh.at[pl.ds(r0, _WB)], wb)
            pltpu.sync_copy(deg_sh.at[pl.ds(r0 // _L, _WB // _L)],
                            degz.at[pl.ds(0, _WB // _L)])

            def rbody(g, _):
                rcpv = 1.0 / (degz[g, :] + 1e-8)
                for k in range(_L):
                    rcp = jnp.broadcast_to(rcpv[k], (_L,))
                    i = g * _L + k
                    for jj in range(nvec):
                        rows0[i, pl.ds(jj * _L, _L)] = (
                            rows0[i, pl.ds(jj * _L, _L)] * rcp)
                return 0
            lax.fori_loop(0, _WB // _L, rbody, 0)
            pltpu.sync_copy(wb, out_hbm.at[c, pl.ds(r0, _WB)])
            return 0
        lax.fori_loop(0, nwb, wbody, 0)
        plsc.subcore_barrier()


def _sc_aggregate(h, edges2, mbase):
    N, D = h.shape
    E = edges2.shape[1] * edges2.shape[2]
    info = plsc.get_sparse_core_info()
    nc, ns = info.num_cores, info.num_subcores
    body = functools.partial(_sc_agg_body, nc, ns, N, D, E, mbase)
    mesh = plsc.VectorSubcoreMesh(core_axis_name="c", subcore_axis_name="s")
    f = pl.kernel(
        body,
        out_type=jax.ShapeDtypeStruct((nc, N, D), jnp.float32),
        mesh=mesh,
        scratch_types=[
            pltpu.VMEM((2, _G, _K), jnp.int32),
            pltpu.VMEM((2, _G, _K), jnp.int32),
            pltpu.VMEM((_K, D), jnp.float32),
            pltpu.VMEM((_K, D), jnp.float32),
            pltpu.VMEM((640, _L), jnp.float32),
            pltpu.VMEM((5, _K), jnp.int32),
            pltpu.VMEM((40, _L), jnp.float32),
            pltpu.VMEM_SHARED((N, D), jnp.float32),
            pltpu.VMEM_SHARED((640, _L), jnp.float32),
            pltpu.SemaphoreType.DMA,
            pltpu.SemaphoreType.DMA,
            pltpu.SemaphoreType.DMA,
            pltpu.SemaphoreType.DMA,
            pltpu.SemaphoreType.DMA,
            pltpu.SemaphoreType.DMA,
        ],
        compiler_params=pltpu.CompilerParams(use_tc_tiling_on_sc=False,
                                             needs_layout_passes=False),
    )
    return f(h, edges2)


def _t1_body(macc_ref, w_ref, a_ref, fcw_ref, fcb_ref, s_ref, *, mb):
    m = pl.program_id(0)
    n = pl.program_id(1)
    x = macc_ref[0]
    w = w_ref[0]
    y = lax.dot_general(x, w, (((1,), (1,)), ((), ())),
                        preferred_element_type=jnp.float32)
    a = a_ref[mb + m]
    e = jnp.where(y >= 0.0, y, a * y)
    t = jnp.tanh(lax.dot_general(e, fcw_ref[...], (((1,), (1,)), ((), ())),
                                 preferred_element_type=jnp.float32)
                 + fcb_ref[...])

    @pl.when(n == 0)
    def _():
        s_ref[...] = jnp.zeros_like(s_ref)
    s_ref[0, 0] += jnp.sum(t, axis=0)


def _t2_body(n_total, sa_ref, sb_ref, att_ref, ma_ref, mb_ref, w_ref, a_ref,
             z_ref):
    sp = jnp.concatenate([sa_ref[...], sb_ref[...]], axis=0)[:, 0, :]
    sp = sp * (1.0 / n_total)
    logits = jnp.sum(sp * att_ref[...], axis=1)
    mx = jnp.max(logits)
    ew = jnp.exp(logits - mx)
    beta = ew / jnp.sum(ew)
    acc = jnp.zeros_like(z_ref)
    for m in range(4):
        src = ma_ref if m < 2 else mb_ref
        y = lax.dot_general(src[m % 2], w_ref[m], (((1,), (1,)), ((), ())),
                            preferred_element_type=jnp.float32)
        e = jnp.where(y >= 0.0, y, a_ref[m] * y)
        acc = acc + beta[m] * e
    z_ref[...] = acc


def _t1(macc, W_agg, prelu_a, fc_W, fc_b2, mb, nb, bn, N, D):
    return pl.pallas_call(
        functools.partial(_t1_body, mb=mb),
        grid=(2, nb),
        in_specs=[
            pl.BlockSpec((1, bn, D), lambda m, n: (m, n, 0)),
            pl.BlockSpec((1, D, D), lambda m, n, _mb=mb: (m + _mb, 0, 0)),
            pl.BlockSpec(memory_space=pltpu.SMEM),
            pl.BlockSpec((D, D), lambda m, n: (0, 0)),
            pl.BlockSpec((1, D), lambda m, n: (0, 0)),
        ],
        out_specs=pl.BlockSpec((1, 1, D), lambda m, n: (m, 0, 0)),
        out_shape=jax.ShapeDtypeStruct((2, 1, D), jnp.float32),
    )(macc, W_agg, prelu_a, fc_W, fc_b2)


def kernel(h, edge_indices, W_agg, prelu_a, fc_W, fc_b, att):
    N, D = h.shape
    P = edge_indices.shape[0]
    E = edge_indices.shape[2]
    edges2 = edge_indices.reshape(2 * P, E // _K, _K)

    macc_a = _sc_aggregate(h, edges2, 0)
    macc_b = _sc_aggregate(h, edges2, 2)

    nb = 10
    bn = N // nb
    fc_b2 = fc_b.reshape(1, D)
    s_a = _t1(macc_a, W_agg, prelu_a, fc_W, fc_b2, 0, nb, bn, N, D)
    s_b = _t1(macc_b, W_agg, prelu_a, fc_W, fc_b2, 2, nb, bn, N, D)

    z = pl.pallas_call(
        functools.partial(_t2_body, N),
        grid=(nb,),
        in_specs=[
            pl.BlockSpec((2, 1, D), lambda n: (0, 0, 0)),
            pl.BlockSpec((2, 1, D), lambda n: (0, 0, 0)),
            pl.BlockSpec((1, D), lambda n: (0, 0)),
            pl.BlockSpec((2, bn, D), lambda n: (0, n, 0)),
            pl.BlockSpec((2, bn, D), lambda n: (0, n, 0)),
            pl.BlockSpec((P, D, D), lambda n: (0, 0, 0)),
            pl.BlockSpec(memory_space=pltpu.SMEM),
        ],
        out_specs=pl.BlockSpec((bn, D), lambda n: (n, 0)),
        out_shape=jax.ShapeDtypeStruct((N, D), jnp.float32),
    )(s_a, s_b, att, macc_a, macc_b, W_agg, prelu_a)
    return z
